# baseline (speedup 1.0000x reference)
import functools

import jax
import jax.numpy as jnp
from jax import lax
from jax.experimental import pallas as pl
from jax.experimental.pallas import tpu as pltpu
from jax.experimental.pallas import tpu_sc as plsc

N_NODES = 10000
N_EDGES = 320000
D = 128
NUM_GRAPHS = 128

NC = 2
NS = 16
CHUNK = 40
EDGES_PER_TILE = N_EDGES // (NC * NS)
CHUNKS_PER_TILE = EDGES_PER_TILE // CHUNK
ROW_CHUNKS = N_NODES // CHUNK
NBUF = 4
_MAIN_ITERS = CHUNKS_PER_TILE // NBUF
_EPI = list(range(_MAIN_ITERS * NBUF, CHUNKS_PER_TILE))


def _sc_aggregate_body(x_hbm, src_hbm, dst_hbm, out_hbm,
                       accum, srcv, dstv,
                       r0, r1, r2, r3, t0, t1, t2, t3,
                       g0, g1, g2, g3, s0, s1, s2, s3):
  bufs = (r0, r1, r2, r3)
  stg = (t0, t1, t2, t3)
  sg = (g0, g1, g2, g3)
  ss = (s0, s1, s2, s3)
  c = lax.axis_index("c")
  s = lax.axis_index("s")
  wid = c * NS + s

  pltpu.sync_copy(src_hbm.at[pl.ds(wid * EDGES_PER_TILE, EDGES_PER_TILE)], srcv)
  pltpu.sync_copy(dst_hbm.at[pl.ds(wid * EDGES_PER_TILE, EDGES_PER_TILE)],
                  dstv.at[pl.ds(0, EDGES_PER_TILE)])

  def gather(k, b):
    return pltpu.async_copy(
        x_hbm.at[srcv.at[pl.ds(k * CHUNK, CHUNK)]], bufs[b], sg[b])

  def wait_gather(k, b):
    pltpu.make_async_copy(
        x_hbm.at[srcv.at[pl.ds(k * CHUNK, CHUNK)]], bufs[b], sg[b]).wait()

  def scatter(k, b):
    for j in range((CHUNK + 15) // 16):
      stg[b][0, pl.ds(j * 16, 16)] = dstv[pl.ds(k * CHUNK + j * 16, 16)]
    return pltpu.async_copy(
        bufs[b], accum.at[stg[b].at[0, pl.ds(0, CHUNK)]], ss[b], add=True)

  def wait_scatter(k, b):
    pltpu.make_async_copy(
        bufs[b], accum.at[stg[b].at[0, pl.ds(0, CHUNK)]], ss[b]).wait()

  def zrow(i, _):
    for j in range(D // 16):
      r0[i, pl.ds(j * 16, 16)] = jnp.zeros((16,), jnp.float32)
    return 0
  lax.fori_loop(0, CHUNK, zrow, 0)

  def zchunk(t, _):
    k = s + t * NS
    @pl.when(k < ROW_CHUNKS)
    def _():
      pltpu.sync_copy(r0, accum.at[pl.ds(k * CHUNK, CHUNK)])
    return 0
  lax.fori_loop(0, (ROW_CHUNKS + NS - 1) // NS, zchunk, 0)
  plsc.subcore_barrier()

  GL = NBUF - 2
  for b in range(GL):
    gather(b, b)

  def step(k, b):
    b2 = (b + GL) % NBUF
    @pl.when(k >= NBUF - GL)
    def _():
      wait_scatter(k - (NBUF - GL), b2)
    @pl.when(k + GL < CHUNKS_PER_TILE)
    def _():
      gather(k + GL, b2)
    wait_gather(k, b)
    scatter(k, b)

  def body(t, _):
    for b in range(NBUF):
      step(t * NBUF + b, b)
    return 0
  lax.fori_loop(0, _MAIN_ITERS, body, 0)

  for k in _EPI:
    step(k, k % NBUF)
  for k in range(CHUNKS_PER_TILE - (NBUF - GL), CHUNKS_PER_TILE):
    wait_scatter(k, k % NBUF)
  plsc.subcore_barrier()

  def wchunk(t, _):
    k = s + t * NS
    @pl.when(k < ROW_CHUNKS)
    def _():
      pltpu.sync_copy(accum.at[pl.ds(k * CHUNK, CHUNK)],
                      out_hbm.at[pl.ds(c * N_NODES + k * CHUNK, CHUNK)])
    return 0
  lax.fori_loop(0, (ROW_CHUNKS + NS - 1) // NS, wchunk, 0)


@functools.cache
def _sc_aggregate():
  return pl.kernel(
      _sc_aggregate_body,
      out_type=jax.ShapeDtypeStruct((NC * N_NODES, D), jnp.float32),
      mesh=plsc.VectorSubcoreMesh(core_axis_name="c", subcore_axis_name="s",
                                  num_cores=NC, num_subcores=NS),
      scratch_types=(
          [pltpu.VMEM_SHARED((N_NODES, D), jnp.float32),
           pltpu.VMEM((EDGES_PER_TILE,), jnp.int32),
           pltpu.VMEM((EDGES_PER_TILE + 16,), jnp.int32)]
          + [pltpu.VMEM((CHUNK, D), jnp.float32)] * NBUF
          + [pltpu.VMEM((1, 128), jnp.int32)] * NBUF
          + [pltpu.SemaphoreType.DMA] * (2 * NBUF)
      ),
  )


def _mlp_tc_body(x_ref, parts_ref, w1_ref, b1_ref, g_ref, be_ref,
                 w2_ref, b2_ref, o_ref):
  h0 = x_ref[...] + parts_ref[0:N_NODES, :] + parts_ref[N_NODES:2 * N_NODES, :]
  h = jnp.dot(h0, w1_ref[...], preferred_element_type=jnp.float32) + b1_ref[...]
  mean = jnp.mean(h, axis=0, keepdims=True)
  var = jnp.mean((h - mean) * (h - mean), axis=0, keepdims=True)
  hn = (h - mean) * lax.rsqrt(var + 1e-5) * g_ref[...] + be_ref[...]
  hr = jnp.maximum(hn, 0.0)
  h2 = jnp.dot(hr, w2_ref[...], preferred_element_type=jnp.float32) + b2_ref[...]
  o_ref[...] = jnp.maximum(h2, 0.0)


def _mlp_tc(x, parts, p):
  return pl.pallas_call(
      _mlp_tc_body,
      out_shape=jax.ShapeDtypeStruct((N_NODES, D), jnp.float32),
  )(x, parts,
    p['W1'], p['b1'].reshape(1, D), p['gamma'].reshape(1, D),
    p['beta'].reshape(1, D), p['W2'], p['b2'].reshape(1, D))


def _head_tc_body(x_ref, batch_ref, w1_ref, b1_ref, w2_ref, b2_ref, o_ref):
  onehot = (batch_ref[...] == lax.broadcasted_iota(jnp.int32, (1, NUM_GRAPHS), 1)
            ).astype(jnp.float32)
  pooled = lax.dot_general(onehot, x_ref[...], (((0,), (0,)), ((), ())),
                           preferred_element_type=jnp.float32)
  h = jnp.maximum(
      jnp.dot(pooled, w1_ref[...], preferred_element_type=jnp.float32)
      + b1_ref[...], 0.0)
  logits = jnp.dot(h, w2_ref[...], preferred_element_type=jnp.float32) + b2_ref[...]
  m = jnp.max(logits, axis=1, keepdims=True)
  e = jnp.exp(logits - m)
  o_ref[...] = e / jnp.sum(e, axis=1, keepdims=True)


def _head_tc(x, batch, hp):
  out_dim = hp['b2'].shape[0]
  return pl.pallas_call(
      _head_tc_body,
      out_shape=jax.ShapeDtypeStruct((NUM_GRAPHS, out_dim), jnp.float32),
  )(x, batch.reshape(N_NODES, 1), hp['W1'], hp['b1'].reshape(1, D),
    hp['W2'], hp['b2'].reshape(1, out_dim))


def kernel(x, edge_index, batch, params):
  src = edge_index[0]
  dst = edge_index[1]
  for l in range(3):
    parts = _sc_aggregate()(x, src, dst)
    x = _mlp_tc(x, parts, params['conv%d' % l])
  return _head_tc(x, batch, params['head'])


# speedup vs baseline: 1.0731x; 1.0731x over previous
import functools

import jax
import jax.numpy as jnp
from jax import lax
from jax.experimental import pallas as pl
from jax.experimental.pallas import tpu as pltpu
from jax.experimental.pallas import tpu_sc as plsc

N_NODES = 10000
N_EDGES = 320000
D = 128
NUM_GRAPHS = 128

NC = 2
NS = 16
CHUNK = 40
EDGES_PER_TILE = N_EDGES // (NC * NS)
CHUNKS_PER_TILE = EDGES_PER_TILE // CHUNK
ROW_CHUNKS = N_NODES // CHUNK
NBUF = 4
_MAIN_ITERS = CHUNKS_PER_TILE // NBUF
_EPI = list(range(_MAIN_ITERS * NBUF, CHUNKS_PER_TILE))


def _sc_aggregate_body(x_hbm, src_hbm, dst_hbm, out_hbm,
                       accum, srcv, dstv,
                       r0, r1, r2, r3, t0, t1, t2, t3,
                       g0, g1, g2, g3, s0, s1, s2, s3):
  bufs = (r0, r1, r2, r3)
  stg = (t0, t1, t2, t3)
  sg = (g0, g1, g2, g3)
  ss = (s0, s1, s2, s3)
  c = lax.axis_index("c")
  s = lax.axis_index("s")
  wid = c * NS + s

  pltpu.sync_copy(src_hbm.at[pl.ds(wid * EDGES_PER_TILE, EDGES_PER_TILE)], srcv)
  pltpu.sync_copy(dst_hbm.at[pl.ds(wid * EDGES_PER_TILE, EDGES_PER_TILE)],
                  dstv.at[pl.ds(0, EDGES_PER_TILE)])

  def gather(k, b):
    return pltpu.async_copy(
        x_hbm.at[srcv.at[pl.ds(k * CHUNK, CHUNK)]], bufs[b], sg[b])

  def wait_gather(k, b):
    pltpu.make_async_copy(
        x_hbm.at[srcv.at[pl.ds(k * CHUNK, CHUNK)]], bufs[b], sg[b]).wait()

  def scatter(k, b):
    for j in range((CHUNK + 15) // 16):
      stg[b][0, pl.ds(j * 16, 16)] = dstv[pl.ds(k * CHUNK + j * 16, 16)]
    return pltpu.async_copy(
        bufs[b], accum.at[stg[b].at[0, pl.ds(0, CHUNK)]], ss[b], add=True)

  def wait_scatter(k, b):
    pltpu.make_async_copy(
        bufs[b], accum.at[stg[b].at[0, pl.ds(0, CHUNK)]], ss[b]).wait()

  def zrow(i, _):
    for j in range(D // 16):
      r0[i, pl.ds(j * 16, 16)] = jnp.zeros((16,), jnp.float32)
    return 0
  lax.fori_loop(0, CHUNK, zrow, 0)

  def zchunk(t, _):
    k = s + t * NS
    @pl.when(k < ROW_CHUNKS)
    def _():
      pltpu.sync_copy(r0, accum.at[pl.ds(k * CHUNK, CHUNK)])
    return 0
  lax.fori_loop(0, (ROW_CHUNKS + NS - 1) // NS, zchunk, 0)
  plsc.subcore_barrier()

  for b in range(NBUF):
    gather(b, b)

  def step(k, b):
    wait_gather(k, b)
    scatter(k, b)
    @pl.when(k + NBUF < CHUNKS_PER_TILE)
    def _():
      wait_scatter(k, b)
      gather(k + NBUF, b)

  def body(t, _):
    for b in range(NBUF):
      step(t * NBUF + b, b)
    return 0
  lax.fori_loop(0, _MAIN_ITERS, body, 0)

  for k in _EPI:
    step(k, k % NBUF)
  for k in range(CHUNKS_PER_TILE - NBUF, CHUNKS_PER_TILE):
    wait_scatter(k, k % NBUF)
  plsc.subcore_barrier()

  def wchunk(t, _):
    k = s + t * NS
    @pl.when(k < ROW_CHUNKS)
    def _():
      pltpu.sync_copy(accum.at[pl.ds(k * CHUNK, CHUNK)],
                      out_hbm.at[pl.ds(c * N_NODES + k * CHUNK, CHUNK)])
    return 0
  lax.fori_loop(0, (ROW_CHUNKS + NS - 1) // NS, wchunk, 0)


@functools.cache
def _sc_aggregate():
  return pl.kernel(
      _sc_aggregate_body,
      out_type=jax.ShapeDtypeStruct((NC * N_NODES, D), jnp.float32),
      mesh=plsc.VectorSubcoreMesh(core_axis_name="c", subcore_axis_name="s",
                                  num_cores=NC, num_subcores=NS),
      scratch_types=(
          [pltpu.VMEM_SHARED((N_NODES, D), jnp.float32),
           pltpu.VMEM((EDGES_PER_TILE,), jnp.int32),
           pltpu.VMEM((EDGES_PER_TILE + 16,), jnp.int32)]
          + [pltpu.VMEM((CHUNK, D), jnp.float32)] * NBUF
          + [pltpu.VMEM((1, 128), jnp.int32)] * NBUF
          + [pltpu.SemaphoreType.DMA] * (2 * NBUF)
      ),
  )


def _mlp_tc_body(x_ref, parts_ref, w1_ref, b1_ref, g_ref, be_ref,
                 w2_ref, b2_ref, o_ref):
  h0 = x_ref[...] + parts_ref[0:N_NODES, :] + parts_ref[N_NODES:2 * N_NODES, :]
  h = jnp.dot(h0, w1_ref[...], preferred_element_type=jnp.float32) + b1_ref[...]
  mean = jnp.mean(h, axis=0, keepdims=True)
  var = jnp.mean((h - mean) * (h - mean), axis=0, keepdims=True)
  hn = (h - mean) * lax.rsqrt(var + 1e-5) * g_ref[...] + be_ref[...]
  hr = jnp.maximum(hn, 0.0)
  h2 = jnp.dot(hr, w2_ref[...], preferred_element_type=jnp.float32) + b2_ref[...]
  o_ref[...] = jnp.maximum(h2, 0.0)


def _mlp_tc(x, parts, p):
  return pl.pallas_call(
      _mlp_tc_body,
      out_shape=jax.ShapeDtypeStruct((N_NODES, D), jnp.float32),
  )(x, parts,
    p['W1'], p['b1'].reshape(1, D), p['gamma'].reshape(1, D),
    p['beta'].reshape(1, D), p['W2'], p['b2'].reshape(1, D))


def _mlp_head_tc_body(x_ref, parts_ref, w1_ref, b1_ref, g_ref, be_ref,
                      w2_ref, b2_ref, batch_ref, hw1_ref, hb1_ref,
                      hw2_ref, hb2_ref, o_ref):
  h0 = x_ref[...] + parts_ref[0:N_NODES, :] + parts_ref[N_NODES:2 * N_NODES, :]
  h = jnp.dot(h0, w1_ref[...], preferred_element_type=jnp.float32) + b1_ref[...]
  mean = jnp.mean(h, axis=0, keepdims=True)
  var = jnp.mean((h - mean) * (h - mean), axis=0, keepdims=True)
  hn = (h - mean) * lax.rsqrt(var + 1e-5) * g_ref[...] + be_ref[...]
  hr = jnp.maximum(hn, 0.0)
  h2 = jnp.dot(hr, w2_ref[...], preferred_element_type=jnp.float32) + b2_ref[...]
  x3 = jnp.maximum(h2, 0.0)
  onehot = (batch_ref[...] == lax.broadcasted_iota(jnp.int32, (1, NUM_GRAPHS), 1)
            ).astype(jnp.float32)
  pooled = lax.dot_general(onehot, x3, (((0,), (0,)), ((), ())),
                           preferred_element_type=jnp.float32)
  hh = jnp.maximum(
      jnp.dot(pooled, hw1_ref[...], preferred_element_type=jnp.float32)
      + hb1_ref[...], 0.0)
  logits = (jnp.dot(hh, hw2_ref[...], preferred_element_type=jnp.float32)
            + hb2_ref[...])
  m = jnp.max(logits, axis=1, keepdims=True)
  e = jnp.exp(logits - m)
  o_ref[...] = e / jnp.sum(e, axis=1, keepdims=True)


def _mlp_head_tc(x, parts, p, batch, hp):
  out_dim = hp['b2'].shape[0]
  return pl.pallas_call(
      _mlp_head_tc_body,
      out_shape=jax.ShapeDtypeStruct((NUM_GRAPHS, out_dim), jnp.float32),
  )(x, parts,
    p['W1'], p['b1'].reshape(1, D), p['gamma'].reshape(1, D),
    p['beta'].reshape(1, D), p['W2'], p['b2'].reshape(1, D),
    batch.reshape(N_NODES, 1), hp['W1'], hp['b1'].reshape(1, D),
    hp['W2'], hp['b2'].reshape(1, out_dim))


def kernel(x, edge_index, batch, params):
  src = edge_index[0]
  dst = edge_index[1]
  for l in range(2):
    parts = _sc_aggregate()(x, src, dst)
    x = _mlp_tc(x, parts, params['conv%d' % l])
  parts = _sc_aggregate()(x, src, dst)
  return _mlp_head_tc(x, parts, params['conv2'], batch, params['head'])


# speedup vs baseline: 1.1800x; 1.0996x over previous
import functools

import jax
import jax.numpy as jnp
from jax import lax
from jax.experimental import pallas as pl
from jax.experimental.pallas import tpu as pltpu
from jax.experimental.pallas import tpu_sc as plsc

N_NODES = 10000
N_EDGES = 320000
D = 128
NUM_GRAPHS = 128

NC = 2
NS = 16
CHUNK = 40
EDGES_PER_TILE = N_EDGES // (NC * NS)
CHUNKS_PER_TILE = EDGES_PER_TILE // CHUNK
ROW_CHUNKS = N_NODES // CHUNK
NBUF = 5
_MAIN_ITERS = CHUNKS_PER_TILE // NBUF
_EPI = list(range(_MAIN_ITERS * NBUF, CHUNKS_PER_TILE))
WSTRIPE = 640


def _sc_aggregate_body(x_hbm, src_hbm, dst_hbm, out_hbm,
                       accum, srcv, dstv,
                       r0, r1, r2, r3, r4, t0, t1, t2, t3, t4,
                       g0, g1, g2, g3, g4, s0, s1, s2, s3, s4, sz):
  bufs = (r0, r1, r2, r3, r4)
  stg = (t0, t1, t2, t3, t4)
  sg = (g0, g1, g2, g3, g4)
  ss = (s0, s1, s2, s3, s4)
  c = lax.axis_index("c")
  s = lax.axis_index("s")
  wid = c * NS + s

  pltpu.sync_copy(src_hbm.at[pl.ds(wid * EDGES_PER_TILE, EDGES_PER_TILE)], srcv)
  pltpu.sync_copy(dst_hbm.at[pl.ds(wid * EDGES_PER_TILE, EDGES_PER_TILE)],
                  dstv.at[pl.ds(0, EDGES_PER_TILE)])

  def gather(k, b):
    return pltpu.async_copy(
        x_hbm.at[srcv.at[pl.ds(k * CHUNK, CHUNK)]], bufs[b], sg[b])

  def wait_gather(k, b):
    pltpu.make_async_copy(
        x_hbm.at[srcv.at[pl.ds(k * CHUNK, CHUNK)]], bufs[b], sg[b]).wait()

  def scatter(k, b):
    for j in range((CHUNK + 15) // 16):
      stg[b][0, pl.ds(j * 16, 16)] = dstv[pl.ds(k * CHUNK + j * 16, 16)]
    return pltpu.async_copy(
        bufs[b], accum.at[stg[b].at[0, pl.ds(0, CHUNK)]], ss[b], add=True)

  def wait_scatter(k, b):
    pltpu.make_async_copy(
        bufs[b], accum.at[stg[b].at[0, pl.ds(0, CHUNK)]], ss[b]).wait()

  def zrow(i, _):
    for j in range(D // 16):
      r0[i, pl.ds(j * 16, 16)] = jnp.zeros((16,), jnp.float32)
    return 0
  lax.fori_loop(0, CHUNK, zrow, 0)

  n_z = (ROW_CHUNKS + NS - 1) // NS

  def zissue(t, _):
    k = s + t * NS
    @pl.when(k < ROW_CHUNKS)
    def _():
      pltpu.async_copy(r0, accum.at[pl.ds(k * CHUNK, CHUNK)], sz)
    return 0
  lax.fori_loop(0, n_z, zissue, 0)

  def zwait(t, _):
    k = s + t * NS
    @pl.when(k < ROW_CHUNKS)
    def _():
      pltpu.make_async_copy(r0, accum.at[pl.ds(k * CHUNK, CHUNK)], sz).wait()
    return 0
  lax.fori_loop(0, n_z, zwait, 0)
  plsc.subcore_barrier()

  for b in range(NBUF):
    gather(b, b)

  def step(k, b):
    wait_gather(k, b)
    scatter(k, b)
    @pl.when(k + NBUF < CHUNKS_PER_TILE)
    def _():
      wait_scatter(k, b)
      gather(k + NBUF, b)

  def body(t, _):
    for b in range(NBUF):
      step(t * NBUF + b, b)
    return 0
  lax.fori_loop(0, _MAIN_ITERS, body, 0)

  for k in _EPI:
    step(k, k % NBUF)
  for k in range(CHUNKS_PER_TILE - NBUF, CHUNKS_PER_TILE):
    wait_scatter(k, k % NBUF)
  plsc.subcore_barrier()

  w0 = s * WSTRIPE
  @pl.when(s < NS - 1)
  def _():
    pltpu.sync_copy(accum.at[pl.ds(w0, WSTRIPE)],
                    out_hbm.at[pl.ds(c * N_NODES + w0, WSTRIPE)])
  @pl.when(s == NS - 1)
  def _():
    tail0 = (NS - 1) * WSTRIPE
    tail = N_NODES - tail0
    pltpu.sync_copy(accum.at[pl.ds(tail0, tail)],
                    out_hbm.at[pl.ds(c * N_NODES + tail0, tail)])


@functools.cache
def _sc_aggregate():
  return pl.kernel(
      _sc_aggregate_body,
      out_type=jax.ShapeDtypeStruct((NC * N_NODES, D), jnp.float32),
      mesh=plsc.VectorSubcoreMesh(core_axis_name="c", subcore_axis_name="s",
                                  num_cores=NC, num_subcores=NS),
      scratch_types=(
          [pltpu.VMEM_SHARED((N_NODES, D), jnp.float32),
           pltpu.VMEM((EDGES_PER_TILE,), jnp.int32),
           pltpu.VMEM((EDGES_PER_TILE + 16,), jnp.int32)]
          + [pltpu.VMEM((CHUNK, D), jnp.float32)] * NBUF
          + [pltpu.VMEM((1, 128), jnp.int32)] * NBUF
          + [pltpu.SemaphoreType.DMA] * (2 * NBUF + 1)
      ),
  )


def _mlp_tc_body(x_ref, parts_ref, w1_ref, b1_ref, g_ref, be_ref,
                 w2_ref, b2_ref, o_ref):
  h0 = x_ref[...] + parts_ref[0:N_NODES, :] + parts_ref[N_NODES:2 * N_NODES, :]
  h = jnp.dot(h0, w1_ref[...], preferred_element_type=jnp.float32) + b1_ref[...]
  mean = jnp.mean(h, axis=0, keepdims=True)
  var = jnp.mean((h - mean) * (h - mean), axis=0, keepdims=True)
  hn = (h - mean) * lax.rsqrt(var + 1e-5) * g_ref[...] + be_ref[...]
  hr = jnp.maximum(hn, 0.0)
  h2 = jnp.dot(hr, w2_ref[...], preferred_element_type=jnp.float32) + b2_ref[...]
  o_ref[...] = jnp.maximum(h2, 0.0)


def _mlp_tc(x, parts, p):
  return pl.pallas_call(
      _mlp_tc_body,
      out_shape=jax.ShapeDtypeStruct((N_NODES, D), jnp.float32),
  )(x, parts,
    p['W1'], p['b1'].reshape(1, D), p['gamma'].reshape(1, D),
    p['beta'].reshape(1, D), p['W2'], p['b2'].reshape(1, D))


def _mlp_head_tc_body(x_ref, parts_ref, w1_ref, b1_ref, g_ref, be_ref,
                      w2_ref, b2_ref, batch_ref, hw1_ref, hb1_ref,
                      hw2_ref, hb2_ref, o_ref):
  h0 = x_ref[...] + parts_ref[0:N_NODES, :] + parts_ref[N_NODES:2 * N_NODES, :]
  h = jnp.dot(h0, w1_ref[...], preferred_element_type=jnp.float32) + b1_ref[...]
  mean = jnp.mean(h, axis=0, keepdims=True)
  var = jnp.mean((h - mean) * (h - mean), axis=0, keepdims=True)
  hn = (h - mean) * lax.rsqrt(var + 1e-5) * g_ref[...] + be_ref[...]
  hr = jnp.maximum(hn, 0.0)
  h2 = jnp.dot(hr, w2_ref[...], preferred_element_type=jnp.float32) + b2_ref[...]
  x3 = jnp.maximum(h2, 0.0)
  onehot = (batch_ref[...] == lax.broadcasted_iota(jnp.int32, (1, NUM_GRAPHS), 1)
            ).astype(jnp.float32)
  pooled = lax.dot_general(onehot, x3, (((0,), (0,)), ((), ())),
                           preferred_element_type=jnp.float32)
  hh = jnp.maximum(
      jnp.dot(pooled, hw1_ref[...], preferred_element_type=jnp.float32)
      + hb1_ref[...], 0.0)
  logits = (jnp.dot(hh, hw2_ref[...], preferred_element_type=jnp.float32)
            + hb2_ref[...])
  m = jnp.max(logits, axis=1, keepdims=True)
  e = jnp.exp(logits - m)
  o_ref[...] = e / jnp.sum(e, axis=1, keepdims=True)


def _mlp_head_tc(x, parts, p, batch, hp):
  out_dim = hp['b2'].shape[0]
  return pl.pallas_call(
      _mlp_head_tc_body,
      out_shape=jax.ShapeDtypeStruct((NUM_GRAPHS, out_dim), jnp.float32),
  )(x, parts,
    p['W1'], p['b1'].reshape(1, D), p['gamma'].reshape(1, D),
    p['beta'].reshape(1, D), p['W2'], p['b2'].reshape(1, D),
    batch.reshape(N_NODES, 1), hp['W1'], hp['b1'].reshape(1, D),
    hp['W2'], hp['b2'].reshape(1, out_dim))


def kernel(x, edge_index, batch, params):
  src = edge_index[0]
  dst = edge_index[1]
  for l in range(2):
    parts = _sc_aggregate()(x, src, dst)
    x = _mlp_tc(x, parts, params['conv%d' % l])
  parts = _sc_aggregate()(x, src, dst)
  return _mlp_head_tc(x, parts, params['conv2'], batch, params['head'])
